# baseline (speedup 1.0000x reference)
import functools

import jax
import jax.numpy as jnp
from jax import lax
from jax.experimental import pallas as pl
from jax.experimental.pallas import tpu as pltpu
from jax.experimental.pallas import tpu_sc as plsc

VOCAB = 1000000
EMBED_DIM = 64
BATCH = 4096
HIST = 200

NUM_CORES = 2
NUM_SUBCORES = 16
NUM_WORKERS = NUM_CORES * NUM_SUBCORES

NPARTS = 4
HIST_PART = HIST // NPARTS
NP = BATCH * HIST_PART
CHUNK = 256
PER_WORKER = NP // NUM_WORKERS
CHUNKS_PER_WORKER = PER_WORKER // CHUNK
NBUF = 5
LAG = 1


def _sc_body(cids_hbm, wids_hbm, table_hbm, out_hbm,
             idx_c, idx_w, rows, sem_gc, sem_gw, sem_s):
    wid = lax.axis_index("s") * NUM_CORES + lax.axis_index("c")
    row0 = wid * CHUNKS_PER_WORKER

    pltpu.sync_copy(cids_hbm.at[pl.ds(row0, CHUNKS_PER_WORKER)], idx_c)
    pltpu.sync_copy(wids_hbm.at[pl.ds(row0, CHUNKS_PER_WORKER)], idx_w)

    def fire_gc(i, b):
        pltpu.async_copy(table_hbm.at[idx_c.at[i]], rows.at[b], sem_gc.at[b])

    def wait_gc(b):
        pltpu.make_async_copy(table_hbm.at[idx_c.at[0]], rows.at[b],
                              sem_gc.at[b]).wait()

    def fire_gw(i, b):
        pltpu.async_copy(table_hbm.at[idx_w.at[i]], rows.at[b], sem_gw.at[b],
                         add=True)

    def wait_gw(b):
        pltpu.make_async_copy(table_hbm.at[idx_w.at[0]], rows.at[b],
                              sem_gw.at[b]).wait()

    def fire_s(i, b):
        pltpu.async_copy(rows.at[b], out_hbm.at[pl.ds((row0 + i) * CHUNK,
                                                      CHUNK)], sem_s.at[b])

    def wait_s(b):
        pltpu.make_async_copy(rows.at[b], out_hbm.at[pl.ds(0, CHUNK)],
                              sem_s.at[b]).wait()

    @pl.loop(0, CHUNKS_PER_WORKER, step=NBUF)
    def _grp(i0):
        for o in range(NBUF):
            i = i0 + o
            b = o

            @pl.when(i >= NBUF)
            def _():
                wait_s(b)

            fire_gc(i, b)

            b1 = (o - LAG) % NBUF

            @pl.when(i >= LAG)
            def _():
                wait_gc(b1)
                fire_gw(i - LAG, b1)

            b2 = (o - 2 * LAG) % NBUF

            @pl.when(i >= 2 * LAG)
            def _():
                wait_gw(b2)
                fire_s(i - 2 * LAG, b2)

    for i in range(CHUNKS_PER_WORKER, CHUNKS_PER_WORKER + 2 * LAG):
        j1 = i - LAG
        if 0 <= j1 < CHUNKS_PER_WORKER:
            wait_gc(j1 % NBUF)
            fire_gw(j1, j1 % NBUF)
        j2 = i - 2 * LAG
        if 0 <= j2 < CHUNKS_PER_WORKER:
            wait_gw(j2 % NBUF)
            fire_s(j2, j2 % NBUF)
    for b in range(NBUF):
        wait_s(b)


@jax.jit
def _run(cids2, wids2, embeddings):
    mesh = plsc.VectorSubcoreMesh(core_axis_name="c", subcore_axis_name="s")
    fn = pl.kernel(
        _sc_body,
        out_type=jax.ShapeDtypeStruct((NP, EMBED_DIM), jnp.float32),
        mesh=mesh,
        scratch_types=[
            pltpu.VMEM((CHUNKS_PER_WORKER, CHUNK), jnp.int32),
            pltpu.VMEM((CHUNKS_PER_WORKER, CHUNK), jnp.int32),
            pltpu.VMEM((NBUF, CHUNK, EMBED_DIM), jnp.float32),
            pltpu.SemaphoreType.DMA((NBUF,)),
            pltpu.SemaphoreType.DMA((NBUF,)),
            pltpu.SemaphoreType.DMA((NBUF,)),
        ],
        compiler_params=pltpu.CompilerParams(use_tc_tiling_on_sc=False),
    )
    return fn(cids2, wids2, embeddings)


def kernel(cids, wids, sids, embeddings):
    del sids
    cids = cids.astype(jnp.int32)
    wids = wids.astype(jnp.int32)
    parts = []
    for k in range(NPARTS):
        sl = slice(k * HIST_PART, (k + 1) * HIST_PART)
        c2 = cids[:, sl].reshape(NP // CHUNK, CHUNK)
        w2 = wids[:, sl].reshape(NP // CHUNK, CHUNK)
        out = _run(c2, w2, embeddings)
        parts.append(out.reshape(BATCH, HIST_PART, EMBED_DIM))
    return jnp.concatenate(parts, axis=1)


# speedup vs baseline: 2.0837x; 2.0837x over previous
import functools

import jax
import jax.numpy as jnp
from jax import lax
from jax.experimental import pallas as pl
from jax.experimental.pallas import tpu as pltpu
from jax.experimental.pallas import tpu_sc as plsc

VOCAB = 1000000
EMBED_DIM = 64
BATCH = 4096
HIST = 200

NUM_CORES = 2
NUM_SUBCORES = 16
NUM_WORKERS = NUM_CORES * NUM_SUBCORES

N = BATCH * HIST
CHUNK = 256
PER_WORKER = N // NUM_WORKERS
CHUNKS_PER_WORKER = PER_WORKER // CHUNK
NBUF = 4
LAG = 1


def _sc_body(cids_hbm, wids_hbm, table_hbm, out_hbm,
             idx_c, idx_w, rows, sem_gc, sem_gw, sem_s):
    wid = lax.axis_index("s") * NUM_CORES + lax.axis_index("c")
    row0 = wid * CHUNKS_PER_WORKER

    pltpu.sync_copy(cids_hbm.at[pl.ds(row0, CHUNKS_PER_WORKER)], idx_c)
    pltpu.sync_copy(wids_hbm.at[pl.ds(row0, CHUNKS_PER_WORKER)], idx_w)

    def fire_gc(i, b):
        pltpu.async_copy(table_hbm.at[idx_c.at[i]], rows.at[b], sem_gc.at[b])

    def wait_gc(b):
        pltpu.make_async_copy(table_hbm.at[idx_c.at[0]], rows.at[b],
                              sem_gc.at[b]).wait()

    def fire_gw(i, b):
        pltpu.async_copy(table_hbm.at[idx_w.at[i]], rows.at[b], sem_gw.at[b],
                         add=True)

    def wait_gw(b):
        pltpu.make_async_copy(table_hbm.at[idx_w.at[0]], rows.at[b],
                              sem_gw.at[b]).wait()

    def fire_s(i, b):
        pltpu.async_copy(rows.at[b], out_hbm.at[pl.ds((row0 + i) * CHUNK,
                                                      CHUNK)], sem_s.at[b])

    def wait_s(b):
        pltpu.make_async_copy(rows.at[b], out_hbm.at[pl.ds(0, CHUNK)],
                              sem_s.at[b]).wait()

    @pl.loop(0, CHUNKS_PER_WORKER, step=NBUF)
    def _grp(i0):
        for o in range(NBUF):
            i = i0 + o
            b = o

            @pl.when(i >= NBUF)
            def _():
                wait_s(b)

            fire_gc(i, b)

            b1 = (o - LAG) % NBUF

            @pl.when(i >= LAG)
            def _():
                wait_gc(b1)
                fire_gw(i - LAG, b1)

            b2 = (o - 2 * LAG) % NBUF

            @pl.when(i >= 2 * LAG)
            def _():
                wait_gw(b2)
                fire_s(i - 2 * LAG, b2)

    for i in range(CHUNKS_PER_WORKER, CHUNKS_PER_WORKER + 2 * LAG):
        j1 = i - LAG
        if 0 <= j1 < CHUNKS_PER_WORKER:
            wait_gc(j1 % NBUF)
            fire_gw(j1, j1 % NBUF)
        j2 = i - 2 * LAG
        if 0 <= j2 < CHUNKS_PER_WORKER:
            wait_gw(j2 % NBUF)
            fire_s(j2, j2 % NBUF)
    for b in range(NBUF):
        wait_s(b)


@jax.jit
def _run(cids2, wids2, table_p):
    mesh = plsc.VectorSubcoreMesh(core_axis_name="c", subcore_axis_name="s")
    fn = pl.kernel(
        _sc_body,
        out_type=jax.ShapeDtypeStruct((N, EMBED_DIM), jnp.float32),
        mesh=mesh,
        scratch_types=[
            pltpu.VMEM((CHUNKS_PER_WORKER, CHUNK), jnp.int32),
            pltpu.VMEM((CHUNKS_PER_WORKER, CHUNK), jnp.int32),
            pltpu.VMEM((NBUF, CHUNK, EMBED_DIM), jnp.float32),
            pltpu.SemaphoreType.DMA((NBUF,)),
            pltpu.SemaphoreType.DMA((NBUF,)),
            pltpu.SemaphoreType.DMA((NBUF,)),
        ],
        compiler_params=pltpu.CompilerParams(use_tc_tiling_on_sc=False),
    )
    return fn(cids2, wids2, table_p)


def kernel(cids, wids, sids, embeddings):
    del sids
    table_p = jnp.pad(embeddings, ((0, 0), (0, 64))).reshape(
        2 * VOCAB, EMBED_DIM)
    cids2 = (cids.astype(jnp.int32) * 2).reshape(N // CHUNK, CHUNK)
    wids2 = (wids.astype(jnp.int32) * 2).reshape(N // CHUNK, CHUNK)
    out = _run(cids2, wids2, table_p)
    return out.reshape(BATCH, HIST, EMBED_DIM)


# speedup vs baseline: 2.7736x; 1.3311x over previous
import functools

import jax
import jax.numpy as jnp
from jax import lax
from jax.experimental import pallas as pl
from jax.experimental.pallas import tpu as pltpu
from jax.experimental.pallas import tpu_sc as plsc

VOCAB = 1000000
EMBED_DIM = 64
BATCH = 4096
HIST = 200

NUM_CORES = 2
NUM_SUBCORES = 16
NUM_WORKERS = NUM_CORES * NUM_SUBCORES

N = BATCH * HIST
CHUNK = 256
PER_WORKER = N // NUM_WORKERS
CHUNKS_PER_WORKER = PER_WORKER // CHUNK
NBUF = 4
LAG = 1


def _sc_body(cids_hbm, wids_hbm, table_hbm, out_hbm,
             idx_c, idx_w, rows, sem_gc, sem_gw, sem_s):
    wid = lax.axis_index("s") * NUM_CORES + lax.axis_index("c")
    row0 = wid * CHUNKS_PER_WORKER

    pltpu.sync_copy(cids_hbm.at[pl.ds(row0, CHUNKS_PER_WORKER)], idx_c)
    pltpu.sync_copy(wids_hbm.at[pl.ds(row0, CHUNKS_PER_WORKER)], idx_w)

    def fire_gc(i, b):
        pltpu.async_copy(table_hbm.at[idx_c.at[i]], rows.at[b], sem_gc.at[b])

    def wait_gc(b):
        pltpu.make_async_copy(table_hbm.at[idx_c.at[0]], rows.at[b],
                              sem_gc.at[b]).wait()

    def fire_gw(i, b):
        pltpu.async_copy(table_hbm.at[idx_w.at[i]], rows.at[b], sem_gw.at[b],
                         add=True)

    def wait_gw(b):
        pltpu.make_async_copy(table_hbm.at[idx_w.at[0]], rows.at[b],
                              sem_gw.at[b]).wait()

    def fire_s(i, b):
        dst = out_hbm.at[pl.ds((row0 + i) * CHUNK, CHUNK),
                         pl.ds(0, EMBED_DIM)]
        pltpu.async_copy(rows.at[b], dst, sem_s.at[b])

    def wait_s(b):
        dst = out_hbm.at[pl.ds(0, CHUNK), pl.ds(0, EMBED_DIM)]
        pltpu.make_async_copy(rows.at[b], dst, sem_s.at[b]).wait()

    @pl.loop(0, CHUNKS_PER_WORKER, step=NBUF)
    def _grp(i0):
        for o in range(NBUF):
            i = i0 + o
            b = o

            @pl.when(i >= NBUF)
            def _():
                wait_s(b)

            fire_gc(i, b)

            b1 = (o - LAG) % NBUF

            @pl.when(i >= LAG)
            def _():
                wait_gc(b1)
                fire_gw(i - LAG, b1)

            b2 = (o - 2 * LAG) % NBUF

            @pl.when(i >= 2 * LAG)
            def _():
                wait_gw(b2)
                fire_s(i - 2 * LAG, b2)

    for i in range(CHUNKS_PER_WORKER, CHUNKS_PER_WORKER + 2 * LAG):
        j1 = i - LAG
        if 0 <= j1 < CHUNKS_PER_WORKER:
            wait_gc(j1 % NBUF)
            fire_gw(j1, j1 % NBUF)
        j2 = i - 2 * LAG
        if 0 <= j2 < CHUNKS_PER_WORKER:
            wait_gw(j2 % NBUF)
            fire_s(j2, j2 % NBUF)
    for b in range(NBUF):
        wait_s(b)


@jax.jit
def _run(cids2, wids2, table_p):
    mesh = plsc.VectorSubcoreMesh(core_axis_name="c", subcore_axis_name="s")
    fn = pl.kernel(
        _sc_body,
        out_type=jax.ShapeDtypeStruct((N, 2 * EMBED_DIM), jnp.float32),
        mesh=mesh,
        scratch_types=[
            pltpu.VMEM((CHUNKS_PER_WORKER, CHUNK), jnp.int32),
            pltpu.VMEM((CHUNKS_PER_WORKER, CHUNK), jnp.int32),
            pltpu.VMEM((NBUF, CHUNK, EMBED_DIM), jnp.float32),
            pltpu.SemaphoreType.DMA((NBUF,)),
            pltpu.SemaphoreType.DMA((NBUF,)),
            pltpu.SemaphoreType.DMA((NBUF,)),
        ],
        compiler_params=pltpu.CompilerParams(use_tc_tiling_on_sc=False),
    )
    return fn(cids2, wids2, table_p)


def kernel(cids, wids, sids, embeddings):
    del sids
    table_p = jnp.pad(embeddings, ((0, 0), (0, 64))).reshape(
        2 * VOCAB, EMBED_DIM)
    cids2 = (cids.astype(jnp.int32) * 2).reshape(N // CHUNK, CHUNK)
    wids2 = (wids.astype(jnp.int32) * 2).reshape(N // CHUNK, CHUNK)
    out = _run(cids2, wids2, table_p)
    return out.reshape(BATCH, HIST, 2 * EMBED_DIM)[:, :, :EMBED_DIM]
